# baseline (speedup 1.0000x reference)
import jax
import jax.numpy as jnp
from jax.experimental import pallas as pl
from jax.experimental.pallas import tpu as pltpu

N_TOK = 32 * 576
DIM = 64
K = 1024
BLK = 6144
N_BLKS = N_TOK // BLK
LOSS_SCALE = 1.25 / (N_TOK * DIM)


def _vq_body(x_ref, tt_ref, tab_ref, out_ref, idx_ref, loss_ref, bias_ref):
    i = pl.program_id(0)
    tt = tt_ref[...]

    @pl.when(i == 0)
    def _():
        bias_ref[...] = -0.5 * jnp.sum(tt * tt, axis=0, keepdims=True)
        loss_ref[0, 0] = 0.0

    xb = x_ref[...]
    dots = jax.lax.dot_general(
        xb, tt, (((1,), (0,)), ((), ())),
        preferred_element_type=jnp.float32)
    score = dots + bias_ref[...]
    max_val = jnp.max(score, axis=1, keepdims=True)
    kio = jax.lax.broadcasted_iota(jnp.int32, (BLK, K), 1).astype(jnp.float32)
    first = jnp.where(score == max_val, kio, jnp.float32(K))
    idx_f = jnp.min(first, axis=1, keepdims=True)
    idx_ref[...] = idx_f.astype(jnp.int32)
    oh = jnp.where(kio == idx_f, 1.0, 0.0)
    out_ref[...] = jax.lax.dot_general(
        oh, tab_ref[...], (((1,), (0,)), ((), ())),
        preferred_element_type=jnp.float32)

    x_sq = jnp.sum(xb * xb)
    loss_ref[0, 0] += (x_sq - 2.0 * jnp.sum(max_val)) * LOSS_SCALE


@jax.jit
def kernel(x, table):
    flat_x = x.reshape(N_TOK, DIM)
    tt = table.T
    out, idx, loss = pl.pallas_call(
        _vq_body,
        grid=(N_BLKS,),
        in_specs=[
            pl.BlockSpec((BLK, DIM), lambda i: (i, 0)),
            pl.BlockSpec((DIM, K), lambda i: (0, 0)),
            pl.BlockSpec((K, DIM), lambda i: (0, 0)),
        ],
        out_specs=[
            pl.BlockSpec((BLK, DIM), lambda i: (i, 0)),
            pl.BlockSpec((BLK, 1), lambda i: (i, 0)),
            pl.BlockSpec(memory_space=pltpu.SMEM),
        ],
        out_shape=[
            jax.ShapeDtypeStruct((N_TOK, DIM), jnp.float32),
            jax.ShapeDtypeStruct((N_TOK, 1), jnp.int32),
            jax.ShapeDtypeStruct((1, 1), jnp.float32),
        ],
        scratch_shapes=[pltpu.VMEM((1, K), jnp.float32)],
    )(flat_x, tt, table)
    return out.reshape(x.shape), loss[0, 0], idx


# speedup vs baseline: 1.0425x; 1.0425x over previous
import jax
import jax.numpy as jnp
from jax.experimental import pallas as pl
from jax.experimental.pallas import tpu as pltpu

N_TOK = 32 * 576
DIM = 64
K = 1024
BLK = 4608
N_BLKS = N_TOK // BLK
LOSS_SCALE = 1.25 / (N_TOK * DIM)
DAUG = 72


def _vq_body(x_ref, tt_ref, tab_ref, out_ref, idx_ref, loss_ref, aug_ref):
    i = pl.program_id(0)

    @pl.when(i == 0)
    def _():
        tt = tt_ref[...]
        aug_ref[:DIM, :] = tt
        aug_ref[DIM:DIM + 1, :] = -0.5 * jnp.sum(tt * tt, axis=0,
                                                 keepdims=True)
        aug_ref[DIM + 1:, :] = jnp.zeros((DAUG - DIM - 1, K), jnp.float32)
        loss_ref[0, 0] = 0.0

    xb = x_ref[...]
    ones = jnp.concatenate(
        [jnp.ones((BLK, 1), jnp.float32),
         jnp.zeros((BLK, DAUG - DIM - 1), jnp.float32)], axis=1)
    xb_aug = jnp.concatenate([xb, ones], axis=1)
    score = jax.lax.dot_general(
        xb_aug, aug_ref[...], (((1,), (0,)), ((), ())),
        preferred_element_type=jnp.float32)
    max_val = jnp.max(score, axis=1, keepdims=True)
    kio = jax.lax.broadcasted_iota(jnp.int32, (BLK, K), 1).astype(jnp.float32)
    first = jnp.where(score == max_val, kio, jnp.float32(K))
    idx_f = jnp.min(first, axis=1, keepdims=True)
    idx_ref[...] = idx_f.astype(jnp.int32)
    oh = jnp.where(kio == idx_f, 1.0, 0.0)
    out_ref[...] = jax.lax.dot_general(
        oh, tab_ref[...], (((1,), (0,)), ((), ())),
        preferred_element_type=jnp.float32)

    x_sq = jnp.sum(xb * xb)
    loss_ref[0, 0] += (x_sq - 2.0 * jnp.sum(max_val)) * LOSS_SCALE


@jax.jit
def kernel(x, table):
    flat_x = x.reshape(N_TOK, DIM)
    tt = table.T
    out, idx, loss = pl.pallas_call(
        _vq_body,
        grid=(N_BLKS,),
        in_specs=[
            pl.BlockSpec((BLK, DIM), lambda i: (i, 0)),
            pl.BlockSpec((DIM, K), lambda i: (0, 0)),
            pl.BlockSpec((K, DIM), lambda i: (0, 0)),
        ],
        out_specs=[
            pl.BlockSpec((BLK, DIM), lambda i: (i, 0)),
            pl.BlockSpec((BLK, 1), lambda i: (i, 0)),
            pl.BlockSpec(memory_space=pltpu.SMEM),
        ],
        out_shape=[
            jax.ShapeDtypeStruct((N_TOK, DIM), jnp.float32),
            jax.ShapeDtypeStruct((N_TOK, 1), jnp.int32),
            jax.ShapeDtypeStruct((1, 1), jnp.float32),
        ],
        scratch_shapes=[pltpu.VMEM((DAUG, K), jnp.float32)],
    )(flat_x, tt, table)
    return out.reshape(x.shape), loss[0, 0], idx
